# baseline (speedup 1.0000x reference)
import functools

import jax
import jax.numpy as jnp
from jax.experimental import pallas as pl
from jax.experimental.pallas import tpu as pltpu


def _merge_matmul_kernel(n_seg, k_bounds, bm, bk, *refs):
    gs = refs[:n_seg]
    xs = refs[n_seg:2 * n_seg]
    ws = refs[2 * n_seg:3 * n_seg]
    o_ref = refs[3 * n_seg]
    acc_ref = refs[3 * n_seg + 1]

    k = pl.program_id(1)

    for s in range(n_seg):
        start, end = k_bounds[s], k_bounds[s + 1]

        @pl.when(k == start)
        def _(s=s):
            acc_ref[...] = jnp.zeros_like(acc_ref)

        @pl.when((k >= start) & (k < end))
        def _(s=s, start=start):
            g = gs[s][...].astype(jnp.bfloat16)
            local_k = k - start
            x_blk = xs[s][pl.ds(local_k * bk, bk), :]
            acc_ref[...] += jnp.dot(g, x_blk,
                                    preferred_element_type=jnp.float32)

        @pl.when(k == end - 1)
        def _(s=s):
            contrib = jax.lax.dot(acc_ref[...], ws[s][...],
                                  precision=jax.lax.Precision.HIGHEST,
                                  preferred_element_type=jnp.float32)
            if s == 0:
                o_ref[...] = contrib
            else:
                o_ref[...] += contrib


def _merged_z(g_list, x_list, w_list, bm, bk):
    m = g_list[0].shape[0]
    n_seg = len(g_list)
    k_steps = [g.shape[1] // bk for g in g_list]
    k_bounds = [0]
    for ks in k_steps:
        k_bounds.append(k_bounds[-1] + ks)
    total_k = k_bounds[-1]

    g_specs = []
    for s, g in enumerate(g_list):
        start, last = k_bounds[s], k_bounds[s + 1] - 1

        def idx(i, k, start=start, last=last):
            return (i, jnp.clip(k - start, 0, last - start))

        g_specs.append(pl.BlockSpec((bm, bk), idx))

    x_specs = [
        pl.BlockSpec(x.shape, lambda i, k: (0, 0)) for x in x_list
    ]
    w_specs = [
        pl.BlockSpec(w.shape, lambda i, k: (0, 0)) for w in w_list
    ]

    out_spec = pl.BlockSpec((bm, 32), lambda i, k: (i, 0))

    kernel_fn = functools.partial(
        _merge_matmul_kernel, n_seg, tuple(k_bounds), bm, bk)

    return pl.pallas_call(
        kernel_fn,
        grid=(m // bm, total_k),
        in_specs=g_specs + x_specs + w_specs,
        out_specs=out_spec,
        out_shape=jax.ShapeDtypeStruct((m, 32), jnp.float32),
        scratch_shapes=[pltpu.VMEM((bm, 32), jnp.float32)],
        compiler_params=pltpu.CompilerParams(
            dimension_semantics=("arbitrary", "arbitrary"),
        ),
    )(*g_list, *x_list, *w_list)


@jax.jit
def kernel(xv, xe, xf, Gv2v, Gv2e, Ge2e, Ge2f, Gf2f, Wv2v, Wve, Wee, Wef, Wff):
    xv16 = xv.astype(jnp.bfloat16)
    xe16 = xe.astype(jnp.bfloat16)
    xf16 = xf.astype(jnp.bfloat16)

    bm, bk = 512, 1024
    zv = _merged_z([Gv2v], [xv16], [Wv2v], bm, bk)
    ze = _merged_z([Gv2e, Ge2e], [xv16, xe16], [Wve, Wee], bm, bk)
    zf = _merged_z([Ge2f, Gf2f], [xe16, xf16], [Wef, Wff], bm, bk)
    return (zv, ze, zf)


# speedup vs baseline: 1.0846x; 1.0846x over previous
import jax
import jax.numpy as jnp
from jax.experimental import pallas as pl
from jax.experimental.pallas import tpu as pltpu


def _gx_t_kernel(x_ref, g_ref, o_ref):
    k = pl.program_id(1)

    @pl.when(k == 0)
    def _():
        o_ref[...] = jnp.zeros_like(o_ref)

    g = g_ref[...].astype(jnp.bfloat16)
    o_ref[...] += jax.lax.dot_general(
        x_ref[...], g,
        dimension_numbers=(((0,), (1,)), ((), ())),
        preferred_element_type=jnp.float32)


def _gx_t(g, x16, bm, bk):
    m, kdim = g.shape
    return pl.pallas_call(
        _gx_t_kernel,
        grid=(m // bm, kdim // bk),
        in_specs=[
            pl.BlockSpec((bk, 32), lambda i, k: (k, 0)),
            pl.BlockSpec((bm, bk), lambda i, k: (i, k)),
        ],
        out_specs=pl.BlockSpec((32, bm), lambda i, k: (0, i)),
        out_shape=jax.ShapeDtypeStruct((32, m), jnp.float32),
        compiler_params=pltpu.CompilerParams(
            dimension_semantics=("arbitrary", "arbitrary"),
        ),
    )(x16, g)


def _w_apply_kernel(tv_ref, te1_ref, te2_ref, tf1_ref, tf2_ref,
                    wv_ref, we1_ref, we2_ref, wf1_ref, wf2_ref,
                    ov_ref, oe_ref, of_ref):
    def wt(w_ref, t_ref):
        return jax.lax.dot_general(
            w_ref[...].astype(jnp.bfloat16),
            t_ref[...].astype(jnp.bfloat16),
            dimension_numbers=(((0,), (0,)), ((), ())),
            preferred_element_type=jnp.float32)

    ov_ref[...] = wt(wv_ref, tv_ref)
    oe_ref[...] = wt(we1_ref, te1_ref) + wt(we2_ref, te2_ref)
    of_ref[...] = wt(wf1_ref, tf1_ref) + wt(wf2_ref, tf2_ref)


def _w_apply(tv, te1, te2, tf1, tf2, wv, we1, we2, wf1, wf2):
    nv = tv.shape[1]
    ne = te1.shape[1]
    nf = tf1.shape[1]
    return pl.pallas_call(
        _w_apply_kernel,
        out_shape=(
            jax.ShapeDtypeStruct((32, nv), jnp.float32),
            jax.ShapeDtypeStruct((32, ne), jnp.float32),
            jax.ShapeDtypeStruct((32, nf), jnp.float32),
        ),
    )(tv, te1, te2, tf1, tf2, wv, we1, we2, wf1, wf2)


@jax.jit
def kernel(xv, xe, xf, Gv2v, Gv2e, Ge2e, Ge2f, Gf2f, Wv2v, Wve, Wee, Wef, Wff):
    xv16 = xv.astype(jnp.bfloat16)
    xe16 = xe.astype(jnp.bfloat16)
    xf16 = xf.astype(jnp.bfloat16)

    bm, bk = 512, 1024
    tv = _gx_t(Gv2v, xv16, bm, bk)
    te1 = _gx_t(Gv2e, xv16, bm, bk)
    te2 = _gx_t(Ge2e, xe16, bm, bk)
    tf1 = _gx_t(Ge2f, xe16, bm, bk)
    tf2 = _gx_t(Gf2f, xf16, bm, bk)

    zvt, zet, zft = _w_apply(tv, te1, te2, tf1, tf2,
                             Wv2v, Wve, Wee, Wef, Wff)
    return (zvt.T, zet.T, zft.T)


# speedup vs baseline: 1.4439x; 1.3312x over previous
import jax
import jax.numpy as jnp
from jax.experimental import pallas as pl
from jax.experimental.pallas import tpu as pltpu


def _gx_t_kernel(x_ref, g_ref, o_ref):
    k = pl.program_id(1)

    @pl.when(k == 0)
    def _():
        o_ref[...] = jnp.zeros_like(o_ref)

    g = g_ref[...].astype(jnp.bfloat16)
    o_ref[...] += jax.lax.dot_general(
        x_ref[...], g,
        dimension_numbers=(((0,), (1,)), ((), ())),
        preferred_element_type=jnp.float32)


def _gx_t(g, x16, bm, bk):
    m, kdim = g.shape
    return pl.pallas_call(
        _gx_t_kernel,
        grid=(m // bm, kdim // bk),
        in_specs=[
            pl.BlockSpec((bk, 32), lambda i, k: (k, 0)),
            pl.BlockSpec((bm, bk), lambda i, k: (i, k)),
        ],
        out_specs=pl.BlockSpec((32, bm), lambda i, k: (0, i)),
        out_shape=jax.ShapeDtypeStruct((32, m), jnp.float32),
        compiler_params=pltpu.CompilerParams(
            dimension_semantics=("arbitrary", "arbitrary"),
        ),
    )(x16, g)


def _w_apply_kernel(tv_ref, te1_ref, te2_ref, tf1_ref, tf2_ref,
                    wv_ref, we1_ref, we2_ref, wf1_ref, wf2_ref,
                    ov_ref, oe_ref, of_ref):
    def wt(w_ref, t_ref):
        return jax.lax.dot_general(
            w_ref[...].astype(jnp.bfloat16),
            t_ref[...].astype(jnp.bfloat16),
            dimension_numbers=(((0,), (0,)), ((), ())),
            preferred_element_type=jnp.float32)

    ov_ref[...] = wt(wv_ref, tv_ref)
    oe_ref[...] = wt(we1_ref, te1_ref) + wt(we2_ref, te2_ref)
    of_ref[...] = wt(wf1_ref, tf1_ref) + wt(wf2_ref, tf2_ref)


def _w_apply(tv, te1, te2, tf1, tf2, wv, we1, we2, wf1, wf2):
    nv = tv.shape[1]
    ne = te1.shape[1]
    nf = tf1.shape[1]
    return pl.pallas_call(
        _w_apply_kernel,
        out_shape=(
            jax.ShapeDtypeStruct((32, nv), jnp.float32),
            jax.ShapeDtypeStruct((32, ne), jnp.float32),
            jax.ShapeDtypeStruct((32, nf), jnp.float32),
        ),
    )(tv, te1, te2, tf1, tf2, wv, we1, we2, wf1, wf2)


@jax.jit
def kernel(xv, xe, xf, Gv2v, Gv2e, Ge2e, Ge2f, Gf2f, Wv2v, Wve, Wee, Wef, Wff):
    xv16 = xv.astype(jnp.bfloat16)
    xe16 = xe.astype(jnp.bfloat16)
    xf16 = xf.astype(jnp.bfloat16)

    bm, bk = 512, 2048
    tv = _gx_t(Gv2v, xv16, bm, bk)
    te1 = _gx_t(Gv2e, xv16, bm, bk)
    te2 = _gx_t(Ge2e, xe16, bm, bk)
    tf1 = _gx_t(Ge2f, xe16, bm, bk)
    tf2 = _gx_t(Gf2f, xf16, bm, bk)

    zvt, zet, zft = _w_apply(tv, te1, te2, tf1, tf2,
                             Wv2v, Wve, Wee, Wef, Wff)
    return (zvt.T, zet.T, zft.T)


# speedup vs baseline: 1.7011x; 1.1782x over previous
import jax
import jax.numpy as jnp
from jax.experimental import pallas as pl
from jax.experimental.pallas import tpu as pltpu


def _gx_t_kernel(x_ref, g_ref, o_ref):
    k = pl.program_id(1)

    @pl.when(k == 0)
    def _():
        o_ref[...] = jnp.zeros_like(o_ref)

    g = g_ref[...].astype(jnp.bfloat16)
    o_ref[...] += jax.lax.dot_general(
        x_ref[...], g,
        dimension_numbers=(((0,), (1,)), ((), ())),
        preferred_element_type=jnp.float32)


def _gx_t(g, x16, bm, bk):
    m, kdim = g.shape
    return pl.pallas_call(
        _gx_t_kernel,
        grid=(m // bm, kdim // bk),
        in_specs=[
            pl.BlockSpec((bk, 32), lambda i, k: (k, 0)),
            pl.BlockSpec((bm, bk), lambda i, k: (i, k)),
        ],
        out_specs=pl.BlockSpec((32, bm), lambda i, k: (0, i)),
        out_shape=jax.ShapeDtypeStruct((32, m), jnp.float32),
        compiler_params=pltpu.CompilerParams(
            dimension_semantics=("arbitrary", "arbitrary"),
        ),
    )(x16, g)


def _w_apply_kernel(tv_ref, te1_ref, te2_ref, tf1_ref, tf2_ref,
                    wv_ref, we1_ref, we2_ref, wf1_ref, wf2_ref,
                    ov_ref, oe_ref, of_ref):
    def wt(w_ref, t_ref):
        return jax.lax.dot_general(
            w_ref[...].astype(jnp.bfloat16),
            t_ref[...].astype(jnp.bfloat16),
            dimension_numbers=(((0,), (0,)), ((), ())),
            preferred_element_type=jnp.float32)

    ov_ref[...] = wt(wv_ref, tv_ref)
    oe_ref[...] = wt(we1_ref, te1_ref) + wt(we2_ref, te2_ref)
    of_ref[...] = wt(wf1_ref, tf1_ref) + wt(wf2_ref, tf2_ref)


def _w_apply(tv, te1, te2, tf1, tf2, wv, we1, we2, wf1, wf2):
    nv = tv.shape[1]
    ne = te1.shape[1]
    nf = tf1.shape[1]
    return pl.pallas_call(
        _w_apply_kernel,
        out_shape=(
            jax.ShapeDtypeStruct((32, nv), jnp.float32),
            jax.ShapeDtypeStruct((32, ne), jnp.float32),
            jax.ShapeDtypeStruct((32, nf), jnp.float32),
        ),
    )(tv, te1, te2, tf1, tf2, wv, we1, we2, wf1, wf2)


@jax.jit
def kernel(xv, xe, xf, Gv2v, Gv2e, Ge2e, Ge2f, Gf2f, Wv2v, Wve, Wee, Wef, Wff):
    xv16 = xv.astype(jnp.bfloat16)
    xe16 = xe.astype(jnp.bfloat16)
    xf16 = xf.astype(jnp.bfloat16)

    bm, bk = 1024, 2048
    tv = _gx_t(Gv2v, xv16, bm, bk)
    te1 = _gx_t(Gv2e, xv16, bm, bk)
    te2 = _gx_t(Ge2e, xe16, bm, bk)
    tf1 = _gx_t(Ge2f, xe16, bm, bk)
    tf2 = _gx_t(Gf2f, xf16, bm, bk)

    zvt, zet, zft = _w_apply(tv, te1, te2, tf1, tf2,
                             Wv2v, Wve, Wee, Wef, Wff)
    return (zvt.T, zet.T, zft.T)
